# baseline (speedup 1.0000x reference)
import functools

import jax
import jax.numpy as jnp
from jax import lax
from jax.experimental import pallas as pl
from jax.experimental.pallas import tpu as pltpu
from jax.experimental.pallas import tpu_sc as plsc

_MODALITY_IDX = 2

_NC = 2
_NS = 16

_BLOCK_ROWS = 2048


def _add_row_kernel(val_ref, row_ref, out_ref):
    out_ref[...] = val_ref[...] + row_ref[...]


def kernel(key, val, device, modality_table):
    b, s, d = val.shape
    n = b * s
    num_mod, _ = modality_table.shape
    mesh = plsc.ScalarSubcoreMesh(axis_name="c", num_cores=1)

    @functools.partial(
        pl.kernel,
        out_type=jax.ShapeDtypeStruct((1, d), jnp.float32),
        mesh=mesh,
    )
    def sc_lookup(table_hbm, row_hbm):
        pltpu.sync_copy(table_hbm.at[pl.ds(_MODALITY_IDX, 1), :], row_hbm)

    row = sc_lookup(modality_table)

    flat = val.reshape(n, d)
    out = pl.pallas_call(
        _add_row_kernel,
        grid=(n // _BLOCK_ROWS,),
        in_specs=[
            pl.BlockSpec((_BLOCK_ROWS, d), lambda i: (i, 0)),
            pl.BlockSpec((1, d), lambda i: (0, 0)),
        ],
        out_specs=pl.BlockSpec((_BLOCK_ROWS, d), lambda i: (i, 0)),
        out_shape=jax.ShapeDtypeStruct((n, d), val.dtype),
    )(flat, row)
    return out.reshape(b, s, d)
